# Initial kernel scaffold; baseline (speedup 1.0000x reference)
import jax
import jax.numpy as jnp
from jax.experimental import pallas as pl


def kernel(x, pos, batch, W, b):
    raise NotImplementedError("write your pallas kernel here")


# speedup vs baseline: 2.1959x; 2.1959x over previous
import jax
import jax.numpy as jnp
from jax.experimental import pallas as pl
from jax.experimental.pallas import tpu as pltpu

N = 320000
D = 128
S = 512
BLK = 2560
NBLK = N // BLK
NEG = float("-inf")


def _fused_kernel(seg_lo_ref, seg_hi_ref, x_ref, pos_ref, ids_ref,
                  w1_ref, w2_ref, b_ref, out_ref):
    i = pl.program_id(0)

    @pl.when(i == 0)
    def _init():
        out_ref[...] = jnp.full((S, D), NEG, dtype=jnp.float32)

    h = (jnp.dot(x_ref[...], w1_ref[...], preferred_element_type=jnp.float32)
         + jnp.dot(pos_ref[...], w2_ref[...], preferred_element_type=jnp.float32)
         + b_ref[...])
    h = jnp.maximum(h, 0.0)

    ids = ids_ref[...]
    lo = seg_lo_ref[i]
    hi = seg_hi_ref[i]

    def body(s, carry):
        mask = ids == s
        m = jnp.max(jnp.where(mask, h, NEG), axis=0, keepdims=True)
        out_ref[pl.ds(s, 1), :] = jnp.maximum(out_ref[pl.ds(s, 1), :], m)
        return carry

    jax.lax.fori_loop(lo, hi + 1, body, 0)


def kernel(x, pos, batch, W, b):
    ids32 = batch.astype(jnp.int32)
    seg_lo = ids32[::BLK]
    seg_hi = ids32[BLK - 1::BLK]
    ids_col = ids32.reshape(N, 1)
    w1 = W[:D]
    w2 = W[D:]
    b2 = b.reshape(1, D)

    grid_spec = pltpu.PrefetchScalarGridSpec(
        num_scalar_prefetch=2,
        grid=(NBLK,),
        in_specs=[
            pl.BlockSpec((BLK, D), lambda i, lo, hi: (i, 0)),
            pl.BlockSpec((BLK, 3), lambda i, lo, hi: (i, 0)),
            pl.BlockSpec((BLK, 1), lambda i, lo, hi: (i, 0)),
            pl.BlockSpec((D, D), lambda i, lo, hi: (0, 0)),
            pl.BlockSpec((3, D), lambda i, lo, hi: (0, 0)),
            pl.BlockSpec((1, D), lambda i, lo, hi: (0, 0)),
        ],
        out_specs=pl.BlockSpec((S, D), lambda i, lo, hi: (0, 0)),
    )

    pooled = pl.pallas_call(
        _fused_kernel,
        grid_spec=grid_spec,
        out_shape=jax.ShapeDtypeStruct((S, D), jnp.float32),
    )(seg_lo, seg_hi, x, pos, ids_col, w1, w2, b2)

    pos_new = jnp.zeros((S, 3), dtype=x.dtype)
    batch_new = jnp.arange(S, dtype=batch.dtype)
    return (pooled, pos_new, batch_new)


# speedup vs baseline: 2.2417x; 1.0209x over previous
import jax
import jax.numpy as jnp
from jax.experimental import pallas as pl
from jax.experimental.pallas import tpu as pltpu

N = 320000
D = 128
S = 512
BLK = 1280
NBLK = N // BLK
NEG = float("-inf")


def _fused_kernel(seg_lo_ref, seg_hi_ref, x_ref, pos_ref, ids_ref,
                  w1_ref, w2_ref, b_ref, out_ref):
    i = pl.program_id(0)

    @pl.when(i == 0)
    def _init():
        out_ref[...] = jnp.full((S, D), NEG, dtype=jnp.float32)

    h = (jnp.dot(x_ref[...], w1_ref[...], preferred_element_type=jnp.float32)
         + jnp.dot(pos_ref[...], w2_ref[...], preferred_element_type=jnp.float32)
         + b_ref[...])
    h = jnp.maximum(h, 0.0)

    ids = ids_ref[...]
    lo = seg_lo_ref[i]
    hi = seg_hi_ref[i]

    def body(s, carry):
        mask = ids == s
        m = jnp.max(jnp.where(mask, h, NEG), axis=0, keepdims=True)
        out_ref[pl.ds(s, 1), :] = jnp.maximum(out_ref[pl.ds(s, 1), :], m)
        return carry

    jax.lax.fori_loop(lo, hi + 1, body, 0)


def kernel(x, pos, batch, W, b):
    ids32 = batch.astype(jnp.int32)
    seg_lo = ids32[::BLK]
    seg_hi = ids32[BLK - 1::BLK]
    ids_col = ids32.reshape(N, 1)
    w1 = W[:D]
    w2 = W[D:]
    b2 = b.reshape(1, D)

    grid_spec = pltpu.PrefetchScalarGridSpec(
        num_scalar_prefetch=2,
        grid=(NBLK,),
        in_specs=[
            pl.BlockSpec((BLK, D), lambda i, lo, hi: (i, 0)),
            pl.BlockSpec((BLK, 3), lambda i, lo, hi: (i, 0)),
            pl.BlockSpec((BLK, 1), lambda i, lo, hi: (i, 0)),
            pl.BlockSpec((D, D), lambda i, lo, hi: (0, 0)),
            pl.BlockSpec((3, D), lambda i, lo, hi: (0, 0)),
            pl.BlockSpec((1, D), lambda i, lo, hi: (0, 0)),
        ],
        out_specs=pl.BlockSpec((S, D), lambda i, lo, hi: (0, 0)),
    )

    pooled = pl.pallas_call(
        _fused_kernel,
        grid_spec=grid_spec,
        out_shape=jax.ShapeDtypeStruct((S, D), jnp.float32),
    )(seg_lo, seg_hi, x, pos, ids_col, w1, w2, b2)

    pos_new = jnp.zeros((S, 3), dtype=x.dtype)
    batch_new = jnp.arange(S, dtype=batch.dtype)
    return (pooled, pos_new, batch_new)


# speedup vs baseline: 2.7572x; 1.2299x over previous
import jax
import jax.numpy as jnp
from jax.experimental import pallas as pl
from jax.experimental.pallas import tpu as pltpu

N = 320000
D = 128
S = 512
BLK = 1280
NBLK = N // BLK
NEG = float("-inf")


def _fused_kernel(seg_lo_ref, seg_hi_ref, starts_ref, x_ref, pos_ref,
                  w1_ref, w2_ref, b_ref, out_ref):
    i = pl.program_id(0)

    @pl.when(i == 0)
    def _init():
        out_ref[...] = jnp.full((S, D), NEG, dtype=jnp.float32)

    h = (jnp.dot(x_ref[...], w1_ref[...], preferred_element_type=jnp.float32)
         + jnp.dot(pos_ref[...], w2_ref[...], preferred_element_type=jnp.float32)
         + b_ref[...])
    h = jnp.maximum(h, 0.0)

    lo = seg_lo_ref[i]
    hi = seg_hi_ref[i]
    base = i * BLK
    row_iota = jax.lax.broadcasted_iota(jnp.int32, (BLK, 1), 0)

    def body(s, carry):
        rs = starts_ref[s] - base
        re = starts_ref[s + 1] - base
        mask = (row_iota >= rs) & (row_iota < re)
        m = jnp.max(jnp.where(mask, h, NEG), axis=0, keepdims=True)
        out_ref[pl.ds(s, 1), :] = jnp.maximum(out_ref[pl.ds(s, 1), :], m)
        return carry

    jax.lax.fori_loop(lo, hi + 1, body, 0)


def kernel(x, pos, batch, W, b):
    ids32 = batch.astype(jnp.int32)
    seg_lo = ids32[::BLK]
    seg_hi = ids32[BLK - 1::BLK]
    starts = jnp.searchsorted(ids32, jnp.arange(S + 1, dtype=jnp.int32),
                              side="left").astype(jnp.int32)
    w1 = W[:D]
    w2 = W[D:]
    b2 = b.reshape(1, D)

    grid_spec = pltpu.PrefetchScalarGridSpec(
        num_scalar_prefetch=3,
        grid=(NBLK,),
        in_specs=[
            pl.BlockSpec((BLK, D), lambda i, *_: (i, 0)),
            pl.BlockSpec((BLK, 3), lambda i, *_: (i, 0)),
            pl.BlockSpec((D, D), lambda i, *_: (0, 0)),
            pl.BlockSpec((3, D), lambda i, *_: (0, 0)),
            pl.BlockSpec((1, D), lambda i, *_: (0, 0)),
        ],
        out_specs=pl.BlockSpec((S, D), lambda i, *_: (0, 0)),
    )

    pooled = pl.pallas_call(
        _fused_kernel,
        grid_spec=grid_spec,
        out_shape=jax.ShapeDtypeStruct((S, D), jnp.float32),
    )(seg_lo, seg_hi, starts, x, pos, w1, w2, b2)

    pos_new = jnp.zeros((S, 3), dtype=x.dtype)
    batch_new = jnp.arange(S, dtype=batch.dtype)
    return (pooled, pos_new, batch_new)


# speedup vs baseline: 2.7695x; 1.0045x over previous
import jax
import jax.numpy as jnp
from jax.experimental import pallas as pl
from jax.experimental.pallas import tpu as pltpu

N = 320000
D = 128
S = 512
BLK = 1280
NBLK = N // BLK
NEG = float("-inf")


def _fused_kernel(x_ref, pos_ref, ids_ref, w1_ref, w2_ref, b_ref, out_ref):
    i = pl.program_id(0)

    @pl.when(i == 0)
    def _init():
        out_ref[...] = jnp.full((S, D), NEG, dtype=jnp.float32)

    h = (jnp.dot(x_ref[...], w1_ref[...], preferred_element_type=jnp.float32)
         + jnp.dot(pos_ref[...], w2_ref[...], preferred_element_type=jnp.float32)
         + b_ref[...])
    h = jnp.maximum(h, 0.0)

    ids = ids_ref[0, 0, :]
    lo = ids_ref[0, 0, 0]
    hi = ids_ref[0, 0, BLK - 1]
    row_iota = jax.lax.broadcasted_iota(jnp.int32, (BLK, 1), 0)

    def body(s, rs):
        re = jnp.sum((ids < s + 1).astype(jnp.int32))
        mask = (row_iota >= rs) & (row_iota < re)
        m = jnp.max(jnp.where(mask, h, NEG), axis=0, keepdims=True)
        out_ref[pl.ds(s, 1), :] = jnp.maximum(out_ref[pl.ds(s, 1), :], m)
        return re

    rs0 = jnp.sum((ids < lo).astype(jnp.int32))
    jax.lax.fori_loop(lo, hi + 1, body, rs0)


def kernel(x, pos, batch, W, b):
    ids3 = batch.astype(jnp.int32).reshape(NBLK, 1, BLK)
    w1 = W[:D]
    w2 = W[D:]
    b2 = b.reshape(1, D)

    pooled = pl.pallas_call(
        _fused_kernel,
        grid=(NBLK,),
        in_specs=[
            pl.BlockSpec((BLK, D), lambda i: (i, 0)),
            pl.BlockSpec((BLK, 3), lambda i: (i, 0)),
            pl.BlockSpec((1, 1, BLK), lambda i: (i, 0, 0)),
            pl.BlockSpec((D, D), lambda i: (0, 0)),
            pl.BlockSpec((3, D), lambda i: (0, 0)),
            pl.BlockSpec((1, D), lambda i: (0, 0)),
        ],
        out_specs=pl.BlockSpec((S, D), lambda i: (0, 0)),
        out_shape=jax.ShapeDtypeStruct((S, D), jnp.float32),
    )(x, pos, ids3, w1, w2, b2)

    pos_new = jnp.zeros((S, 3), dtype=x.dtype)
    batch_new = jnp.arange(S, dtype=batch.dtype)
    return (pooled, pos_new, batch_new)


# speedup vs baseline: 3.2020x; 1.1562x over previous
import jax
import jax.numpy as jnp
from jax.experimental import pallas as pl
from jax.experimental.pallas import tpu as pltpu

N = 320000
D = 128
S = 512
BLK = 1280
NBLK = N // BLK
CH = 128
NCH = N // CH
NEG = float("-inf")


def _fused_kernel(seg_lo_ref, seg_hi_ref, starts_ref, x_ref, pos_ref,
                  w1_ref, w2_ref, b_ref, out_ref):
    i = pl.program_id(0)

    @pl.when(i == 0)
    def _init():
        out_ref[...] = jnp.full((S, D), NEG, dtype=jnp.float32)

    h = (jnp.dot(x_ref[...], w1_ref[...], preferred_element_type=jnp.float32)
         + jnp.dot(pos_ref[...], w2_ref[...], preferred_element_type=jnp.float32)
         + b_ref[...])
    h = jnp.maximum(h, 0.0)

    lo = seg_lo_ref[i]
    hi = seg_hi_ref[i]
    base = i * BLK
    row_iota = jax.lax.broadcasted_iota(jnp.int32, (BLK, 1), 0)

    def body(s, carry):
        rs = starts_ref[s] - base
        re = starts_ref[s + 1] - base
        mask = (row_iota >= rs) & (row_iota < re)
        m = jnp.max(jnp.where(mask, h, NEG), axis=0, keepdims=True)
        out_ref[pl.ds(s, 1), :] = jnp.maximum(out_ref[pl.ds(s, 1), :], m)
        return carry

    jax.lax.fori_loop(lo, hi + 1, body, 0)


def _segment_tables(ids32):
    q = jnp.arange(S + 1, dtype=jnp.int32)
    chunk_last = ids32[CH - 1::CH]
    coarse = jnp.sum(chunk_last[None, :] < q[:, None], axis=1,
                     dtype=jnp.int32)
    p = jnp.minimum(coarse, NCH - 1)
    rows = ids32.reshape(NCH, CH)[p]
    fine = jnp.sum(rows < q[:, None], axis=1, dtype=jnp.int32)
    starts = CH * p + fine

    bases = jnp.arange(NBLK, dtype=jnp.int32) * BLK
    seg_lo = jnp.sum(starts[None, :] <= bases[:, None], axis=1,
                     dtype=jnp.int32) - 1
    seg_hi = jnp.sum(starts[None, :] <= (bases + BLK - 1)[:, None], axis=1,
                     dtype=jnp.int32) - 1
    return seg_lo, seg_hi, starts


def kernel(x, pos, batch, W, b):
    ids32 = batch.astype(jnp.int32)
    seg_lo, seg_hi, starts = _segment_tables(ids32)
    w1 = W[:D]
    w2 = W[D:]
    b2 = b.reshape(1, D)

    grid_spec = pltpu.PrefetchScalarGridSpec(
        num_scalar_prefetch=3,
        grid=(NBLK,),
        in_specs=[
            pl.BlockSpec((BLK, D), lambda i, *_: (i, 0)),
            pl.BlockSpec((BLK, 3), lambda i, *_: (i, 0)),
            pl.BlockSpec((D, D), lambda i, *_: (0, 0)),
            pl.BlockSpec((3, D), lambda i, *_: (0, 0)),
            pl.BlockSpec((1, D), lambda i, *_: (0, 0)),
        ],
        out_specs=pl.BlockSpec((S, D), lambda i, *_: (0, 0)),
    )

    pooled = pl.pallas_call(
        _fused_kernel,
        grid_spec=grid_spec,
        out_shape=jax.ShapeDtypeStruct((S, D), jnp.float32),
    )(seg_lo, seg_hi, starts, x, pos, w1, w2, b2)

    pos_new = jnp.zeros((S, 3), dtype=x.dtype)
    batch_new = jnp.arange(S, dtype=batch.dtype)
    return (pooled, pos_new, batch_new)


# speedup vs baseline: 3.3850x; 1.0571x over previous
import jax
import jax.numpy as jnp
from jax.experimental import pallas as pl
from jax.experimental.pallas import tpu as pltpu

N = 320000
D = 128
S = 512
BLK = 1280
NSTEP = N // (2 * BLK)
CH = 128
NCH = N // CH
NEG = float("-inf")


def _fused_kernel(seg_lo_ref, seg_hi_ref, starts_ref, xa_ref, xb_ref,
                  pos_ref, w1_ref, w2_ref, b_ref, out_ref):
    i = pl.program_id(0)

    @pl.when(i == 0)
    def _init():
        out_ref[...] = jnp.full((S, D), NEG, dtype=jnp.float32)

    w1 = w1_ref[...]
    pb = jnp.dot(pos_ref[...], w2_ref[...],
                 preferred_element_type=jnp.float32) + b_ref[...]
    ha = jnp.maximum(
        jnp.dot(xa_ref[...], w1, preferred_element_type=jnp.float32)
        + pb[:BLK], 0.0)
    hb = jnp.maximum(
        jnp.dot(xb_ref[...], w1, preferred_element_type=jnp.float32)
        + pb[BLK:], 0.0)

    lo = seg_lo_ref[i]
    hi = seg_hi_ref[i]
    base = i * (2 * BLK)
    row_iota = jax.lax.broadcasted_iota(jnp.int32, (BLK, 1), 0)

    def body(s, carry):
        rs = starts_ref[s] - base
        re = starts_ref[s + 1] - base
        mask_a = (row_iota >= rs) & (row_iota < re)
        mask_b = (row_iota >= rs - BLK) & (row_iota < re - BLK)
        m = jnp.maximum(
            jnp.max(jnp.where(mask_a, ha, NEG), axis=0, keepdims=True),
            jnp.max(jnp.where(mask_b, hb, NEG), axis=0, keepdims=True))
        out_ref[pl.ds(s, 1), :] = jnp.maximum(out_ref[pl.ds(s, 1), :], m)
        return carry

    jax.lax.fori_loop(lo, hi + 1, body, 0)


def _segment_tables(ids32):
    q = jnp.arange(S + 1, dtype=jnp.int32)
    chunk_last = ids32[CH - 1::CH]
    coarse = jnp.sum(chunk_last[None, :] < q[:, None], axis=1,
                     dtype=jnp.int32)
    p = jnp.minimum(coarse, NCH - 1)
    rows = ids32.reshape(NCH, CH)[p]
    fine = jnp.sum(rows < q[:, None], axis=1, dtype=jnp.int32)
    starts = CH * p + fine

    bases = jnp.arange(NSTEP, dtype=jnp.int32) * (2 * BLK)
    seg_lo = jnp.sum(starts[None, :] <= bases[:, None], axis=1,
                     dtype=jnp.int32) - 1
    seg_hi = jnp.sum(starts[None, :] <= (bases + 2 * BLK - 1)[:, None],
                     axis=1, dtype=jnp.int32) - 1
    return seg_lo, seg_hi, starts


def kernel(x, pos, batch, W, b):
    ids32 = batch.astype(jnp.int32)
    seg_lo, seg_hi, starts = _segment_tables(ids32)
    w1 = W[:D]
    w2 = W[D:]
    b2 = b.reshape(1, D)

    grid_spec = pltpu.PrefetchScalarGridSpec(
        num_scalar_prefetch=3,
        grid=(NSTEP,),
        in_specs=[
            pl.BlockSpec((BLK, D), lambda i, *_: (2 * i, 0)),
            pl.BlockSpec((BLK, D), lambda i, *_: (2 * i + 1, 0)),
            pl.BlockSpec((2 * BLK, 3), lambda i, *_: (i, 0)),
            pl.BlockSpec((D, D), lambda i, *_: (0, 0)),
            pl.BlockSpec((3, D), lambda i, *_: (0, 0)),
            pl.BlockSpec((1, D), lambda i, *_: (0, 0)),
        ],
        out_specs=pl.BlockSpec((S, D), lambda i, *_: (0, 0)),
    )

    pooled = pl.pallas_call(
        _fused_kernel,
        grid_spec=grid_spec,
        out_shape=jax.ShapeDtypeStruct((S, D), jnp.float32),
    )(seg_lo, seg_hi, starts, x, x, pos, w1, w2, b2)

    pos_new = jnp.zeros((S, 3), dtype=x.dtype)
    batch_new = jnp.arange(S, dtype=batch.dtype)
    return (pooled, pos_new, batch_new)


# speedup vs baseline: 4.5818x; 1.3535x over previous
import jax
import jax.numpy as jnp
from jax.experimental import pallas as pl
from jax.experimental.pallas import tpu as pltpu

N = 320000
D = 128
S = 512
BLKR = 2560
NSTEP = N // BLKR
NCHB = BLKR // 8
CH = 128
NCH = N // CH
NEG = float("-inf")


def _fused_kernel(seg_lo_ref, seg_hi_ref, starts_ref, x_ref, pos_ref,
                  w1_ref, w2_ref, b_ref, out_ref, h_ref):
    i = pl.program_id(0)

    @pl.when(i == 0)
    def _init():
        out_ref[...] = jnp.full((S, D), NEG, dtype=jnp.float32)

    h = (jnp.dot(x_ref[...], w1_ref[...], preferred_element_type=jnp.float32)
         + jnp.dot(pos_ref[...], w2_ref[...], preferred_element_type=jnp.float32)
         + b_ref[...])
    h_ref[...] = jnp.maximum(h, 0.0)

    lo = seg_lo_ref[i]
    hi = seg_hi_ref[i]
    base = i * BLKR
    iota8 = jax.lax.broadcasted_iota(jnp.int32, (8, 1), 0)
    neg8 = jnp.full((8, D), NEG, dtype=jnp.float32)

    def body(s, carry):
        rs = jnp.clip(starts_ref[s] - base, 0, BLKR)
        re = jnp.clip(starts_ref[s + 1] - base, 0, BLKR)

        gl = jnp.minimum(rs // 8, NCHB - 1)
        gr = jnp.minimum(jnp.maximum(re - 1, 0) // 8, NCHB - 1)
        il = iota8 + 8 * gl
        ir = iota8 + 8 * gr
        el = jnp.where((il >= rs) & (il < re), h_ref[pl.ds(8 * gl, 8), :], neg8)
        er = jnp.where((ir >= rs) & (ir < re), h_ref[pl.ds(8 * gr, 8), :], neg8)

        gi0 = (rs + 7) // 8
        gi1 = re // 8
        n4 = jnp.maximum(gi1 - gi0, 0) // 4

        def body4(k, accs):
            g = gi0 + 4 * k
            a0, a1, a2, a3 = accs
            return (jnp.maximum(a0, h_ref[pl.ds(8 * g, 8), :]),
                    jnp.maximum(a1, h_ref[pl.ds(8 * (g + 1), 8), :]),
                    jnp.maximum(a2, h_ref[pl.ds(8 * (g + 2), 8), :]),
                    jnp.maximum(a3, h_ref[pl.ds(8 * (g + 3), 8), :]))

        a0, a1, a2, a3 = jax.lax.fori_loop(0, n4, body4,
                                           (el, er, neg8, neg8))

        def body1(g, acc):
            return jnp.maximum(acc, h_ref[pl.ds(8 * g, 8), :])

        a0 = jax.lax.fori_loop(gi0 + 4 * n4, gi1, body1, a0)

        m8 = jnp.maximum(jnp.maximum(a0, a1), jnp.maximum(a2, a3))
        m = jnp.max(m8, axis=0, keepdims=True)
        out_ref[pl.ds(s, 1), :] = jnp.maximum(out_ref[pl.ds(s, 1), :], m)
        return carry

    jax.lax.fori_loop(lo, hi + 1, body, 0)


def _segment_tables(ids32):
    q = jnp.arange(S + 1, dtype=jnp.int32)
    chunk_last = ids32[CH - 1::CH]
    coarse = jnp.sum(chunk_last[None, :] < q[:, None], axis=1,
                     dtype=jnp.int32)
    p = jnp.minimum(coarse, NCH - 1)
    rows = ids32.reshape(NCH, CH)[p]
    fine = jnp.sum(rows < q[:, None], axis=1, dtype=jnp.int32)
    starts = CH * p + fine

    bases = jnp.arange(NSTEP, dtype=jnp.int32) * BLKR
    seg_lo = jnp.sum(starts[None, :] <= bases[:, None], axis=1,
                     dtype=jnp.int32) - 1
    seg_hi = jnp.sum(starts[None, :] <= (bases + BLKR - 1)[:, None],
                     axis=1, dtype=jnp.int32) - 1
    return seg_lo, seg_hi, starts


def kernel(x, pos, batch, W, b):
    ids32 = batch.astype(jnp.int32)
    seg_lo, seg_hi, starts = _segment_tables(ids32)
    w1 = W[:D]
    w2 = W[D:]
    b2 = b.reshape(1, D)

    grid_spec = pltpu.PrefetchScalarGridSpec(
        num_scalar_prefetch=3,
        grid=(NSTEP,),
        in_specs=[
            pl.BlockSpec((BLKR, D), lambda i, *_: (i, 0)),
            pl.BlockSpec((BLKR, 3), lambda i, *_: (i, 0)),
            pl.BlockSpec((D, D), lambda i, *_: (0, 0)),
            pl.BlockSpec((3, D), lambda i, *_: (0, 0)),
            pl.BlockSpec((1, D), lambda i, *_: (0, 0)),
        ],
        out_specs=pl.BlockSpec((S, D), lambda i, *_: (0, 0)),
        scratch_shapes=[pltpu.VMEM((BLKR, D), jnp.float32)],
    )

    pooled = pl.pallas_call(
        _fused_kernel,
        grid_spec=grid_spec,
        out_shape=jax.ShapeDtypeStruct((S, D), jnp.float32),
    )(seg_lo, seg_hi, starts, x, pos, w1, w2, b2)

    pos_new = jnp.zeros((S, 3), dtype=x.dtype)
    batch_new = jnp.arange(S, dtype=batch.dtype)
    return (pooled, pos_new, batch_new)


# speedup vs baseline: 5.3496x; 1.1676x over previous
import jax
import jax.numpy as jnp
from jax.experimental import pallas as pl
from jax.experimental.pallas import tpu as pltpu

N = 320000
D = 128
S = 512
BLKR = 5120
NSTEP = N // BLKR
NCHB = BLKR // 8
CH = 128
NCH = N // CH
NEG = float("-inf")


def _fused_kernel(seg_lo_ref, seg_hi_ref, starts_ref, x_ref, pos_ref,
                  w1_ref, w2_ref, b_ref, out_ref, h_ref):
    i = pl.program_id(0)

    @pl.when(i == 0)
    def _init():
        out_ref[...] = jnp.full((S, D), NEG, dtype=jnp.float32)

    h = (jnp.dot(x_ref[...], w1_ref[...], preferred_element_type=jnp.float32)
         + jnp.dot(pos_ref[...], w2_ref[...], preferred_element_type=jnp.float32)
         + b_ref[...])
    h_ref[...] = jnp.maximum(h, 0.0)

    lo = seg_lo_ref[i]
    hi = seg_hi_ref[i]
    base = i * BLKR
    iota8 = jax.lax.broadcasted_iota(jnp.int32, (8, 1), 0)
    neg8 = jnp.full((8, D), NEG, dtype=jnp.float32)

    def body(s, carry):
        rs = jnp.clip(starts_ref[s] - base, 0, BLKR)
        re = jnp.clip(starts_ref[s + 1] - base, 0, BLKR)

        gl = jnp.minimum(rs // 8, NCHB - 1)
        gr = jnp.minimum(jnp.maximum(re - 1, 0) // 8, NCHB - 1)
        il = iota8 + 8 * gl
        ir = iota8 + 8 * gr
        el = jnp.where((il >= rs) & (il < re), h_ref[pl.ds(8 * gl, 8), :], neg8)
        er = jnp.where((ir >= rs) & (ir < re), h_ref[pl.ds(8 * gr, 8), :], neg8)

        gi0 = (rs + 7) // 8
        gi1 = re // 8
        n4 = jnp.maximum(gi1 - gi0, 0) // 4

        def body4(k, accs):
            g = gi0 + 4 * k
            a0, a1, a2, a3 = accs
            return (jnp.maximum(a0, h_ref[pl.ds(8 * g, 8), :]),
                    jnp.maximum(a1, h_ref[pl.ds(8 * (g + 1), 8), :]),
                    jnp.maximum(a2, h_ref[pl.ds(8 * (g + 2), 8), :]),
                    jnp.maximum(a3, h_ref[pl.ds(8 * (g + 3), 8), :]))

        a0, a1, a2, a3 = jax.lax.fori_loop(0, n4, body4,
                                           (el, er, neg8, neg8))

        def body1(g, acc):
            return jnp.maximum(acc, h_ref[pl.ds(8 * g, 8), :])

        a0 = jax.lax.fori_loop(gi0 + 4 * n4, gi1, body1, a0)

        m8 = jnp.maximum(jnp.maximum(a0, a1), jnp.maximum(a2, a3))
        m = jnp.max(m8, axis=0, keepdims=True)
        out_ref[pl.ds(s, 1), :] = jnp.maximum(out_ref[pl.ds(s, 1), :], m)
        return carry

    jax.lax.fori_loop(lo, hi + 1, body, 0)


def _segment_tables(ids32):
    q = jnp.arange(S + 1, dtype=jnp.int32)
    chunk_last = ids32[CH - 1::CH]
    coarse = jnp.sum(chunk_last[None, :] < q[:, None], axis=1,
                     dtype=jnp.int32)
    p = jnp.minimum(coarse, NCH - 1)
    rows = ids32.reshape(NCH, CH)[p]
    fine = jnp.sum(rows < q[:, None], axis=1, dtype=jnp.int32)
    starts = CH * p + fine

    bases = jnp.arange(NSTEP, dtype=jnp.int32) * BLKR
    seg_lo = jnp.sum(starts[None, :] <= bases[:, None], axis=1,
                     dtype=jnp.int32) - 1
    seg_hi = jnp.sum(starts[None, :] <= (bases + BLKR - 1)[:, None],
                     axis=1, dtype=jnp.int32) - 1
    return seg_lo, seg_hi, starts


def kernel(x, pos, batch, W, b):
    ids32 = batch.astype(jnp.int32)
    seg_lo, seg_hi, starts = _segment_tables(ids32)
    w1 = W[:D]
    w2 = W[D:]
    b2 = b.reshape(1, D)

    grid_spec = pltpu.PrefetchScalarGridSpec(
        num_scalar_prefetch=3,
        grid=(NSTEP,),
        in_specs=[
            pl.BlockSpec((BLKR, D), lambda i, *_: (i, 0)),
            pl.BlockSpec((BLKR, 3), lambda i, *_: (i, 0)),
            pl.BlockSpec((D, D), lambda i, *_: (0, 0)),
            pl.BlockSpec((3, D), lambda i, *_: (0, 0)),
            pl.BlockSpec((1, D), lambda i, *_: (0, 0)),
        ],
        out_specs=pl.BlockSpec((S, D), lambda i, *_: (0, 0)),
        scratch_shapes=[pltpu.VMEM((BLKR, D), jnp.float32)],
    )

    pooled = pl.pallas_call(
        _fused_kernel,
        grid_spec=grid_spec,
        out_shape=jax.ShapeDtypeStruct((S, D), jnp.float32),
    )(seg_lo, seg_hi, starts, x, pos, w1, w2, b2)

    pos_new = jnp.zeros((S, 3), dtype=x.dtype)
    batch_new = jnp.arange(S, dtype=batch.dtype)
    return (pooled, pos_new, batch_new)


# speedup vs baseline: 5.5049x; 1.0290x over previous
import jax
import jax.numpy as jnp
from jax.experimental import pallas as pl
from jax.experimental.pallas import tpu as pltpu

N = 320000
D = 128
S = 512
BLKR = 6400
NSTEP = N // BLKR
NCHB = BLKR // 8
CH = 128
NCH = N // CH
NEG = float("-inf")


def _fused_kernel(seg_lo_ref, seg_hi_ref, starts_ref, x_ref, pos_ref,
                  w1_ref, w2_ref, b_ref, out_ref, h_ref):
    i = pl.program_id(0)

    @pl.when(i == 0)
    def _init():
        out_ref[...] = jnp.full((S, D), NEG, dtype=jnp.float32)

    h = (jnp.dot(x_ref[...], w1_ref[...], preferred_element_type=jnp.float32)
         + jnp.dot(pos_ref[...], w2_ref[...], preferred_element_type=jnp.float32)
         + b_ref[...])
    h_ref[...] = jnp.maximum(h, 0.0)

    lo = seg_lo_ref[i]
    hi = seg_hi_ref[i]
    base = i * BLKR
    iota8 = jax.lax.broadcasted_iota(jnp.int32, (8, 1), 0)
    neg8 = jnp.full((8, D), NEG, dtype=jnp.float32)

    def body(s, carry):
        rs = jnp.clip(starts_ref[s] - base, 0, BLKR)
        re = jnp.clip(starts_ref[s + 1] - base, 0, BLKR)

        gl = jnp.minimum(rs // 8, NCHB - 1)
        gr = jnp.minimum(jnp.maximum(re - 1, 0) // 8, NCHB - 1)
        il = iota8 + 8 * gl
        ir = iota8 + 8 * gr
        el = jnp.where((il >= rs) & (il < re), h_ref[pl.ds(8 * gl, 8), :], neg8)
        er = jnp.where((ir >= rs) & (ir < re), h_ref[pl.ds(8 * gr, 8), :], neg8)

        gi0 = (rs + 7) // 8
        gi1 = re // 8
        n4 = jnp.maximum(gi1 - gi0, 0) // 4

        def body4(k, accs):
            g = gi0 + 4 * k
            a0, a1, a2, a3 = accs
            return (jnp.maximum(a0, h_ref[pl.ds(8 * g, 8), :]),
                    jnp.maximum(a1, h_ref[pl.ds(8 * (g + 1), 8), :]),
                    jnp.maximum(a2, h_ref[pl.ds(8 * (g + 2), 8), :]),
                    jnp.maximum(a3, h_ref[pl.ds(8 * (g + 3), 8), :]))

        a0, a1, a2, a3 = jax.lax.fori_loop(0, n4, body4,
                                           (el, er, neg8, neg8))

        def body1(g, acc):
            return jnp.maximum(acc, h_ref[pl.ds(8 * g, 8), :])

        a0 = jax.lax.fori_loop(gi0 + 4 * n4, gi1, body1, a0)

        m8 = jnp.maximum(jnp.maximum(a0, a1), jnp.maximum(a2, a3))
        m = jnp.max(m8, axis=0, keepdims=True)
        out_ref[pl.ds(s, 1), :] = jnp.maximum(out_ref[pl.ds(s, 1), :], m)
        return carry

    jax.lax.fori_loop(lo, hi + 1, body, 0)


def _segment_tables(ids32):
    q = jnp.arange(S + 1, dtype=jnp.int32)
    chunk_last = ids32[CH - 1::CH]
    coarse = jnp.sum(chunk_last[None, :] < q[:, None], axis=1,
                     dtype=jnp.int32)
    p = jnp.minimum(coarse, NCH - 1)
    rows = ids32.reshape(NCH, CH)[p]
    fine = jnp.sum(rows < q[:, None], axis=1, dtype=jnp.int32)
    starts = CH * p + fine

    bases = jnp.arange(NSTEP, dtype=jnp.int32) * BLKR
    seg_lo = jnp.sum(starts[None, :] <= bases[:, None], axis=1,
                     dtype=jnp.int32) - 1
    seg_hi = jnp.sum(starts[None, :] <= (bases + BLKR - 1)[:, None],
                     axis=1, dtype=jnp.int32) - 1
    return seg_lo, seg_hi, starts


def kernel(x, pos, batch, W, b):
    ids32 = batch.astype(jnp.int32)
    seg_lo, seg_hi, starts = _segment_tables(ids32)
    w1 = W[:D]
    w2 = W[D:]
    b2 = b.reshape(1, D)

    grid_spec = pltpu.PrefetchScalarGridSpec(
        num_scalar_prefetch=3,
        grid=(NSTEP,),
        in_specs=[
            pl.BlockSpec((BLKR, D), lambda i, *_: (i, 0)),
            pl.BlockSpec((BLKR, 3), lambda i, *_: (i, 0)),
            pl.BlockSpec((D, D), lambda i, *_: (0, 0)),
            pl.BlockSpec((3, D), lambda i, *_: (0, 0)),
            pl.BlockSpec((1, D), lambda i, *_: (0, 0)),
        ],
        out_specs=pl.BlockSpec((S, D), lambda i, *_: (0, 0)),
        scratch_shapes=[pltpu.VMEM((BLKR, D), jnp.float32)],
    )

    pooled = pl.pallas_call(
        _fused_kernel,
        grid_spec=grid_spec,
        out_shape=jax.ShapeDtypeStruct((S, D), jnp.float32),
    )(seg_lo, seg_hi, starts, x, pos, w1, w2, b2)

    pos_new = jnp.zeros((S, 3), dtype=x.dtype)
    batch_new = jnp.arange(S, dtype=batch.dtype)
    return (pooled, pos_new, batch_new)


# speedup vs baseline: 5.8606x; 1.0646x over previous
import jax
import jax.numpy as jnp
from jax.experimental import pallas as pl
from jax.experimental.pallas import tpu as pltpu

N = 320000
D = 128
S = 512
BLKR = 6400
NSTEP = N // BLKR
G = 32
NG = BLKR // G
CH = 128
NCH = N // CH
NEG = float("-inf")


def _fused_kernel(seg_lo_ref, seg_hi_ref, starts_ref, x_ref, pos_ref,
                  w1_ref, w2_ref, b_ref, out_ref, h_ref):
    i = pl.program_id(0)

    @pl.when(i == 0)
    def _init():
        out_ref[...] = jnp.full((S, D), NEG, dtype=jnp.float32)

    h_ref[...] = (
        jnp.dot(x_ref[...], w1_ref[...], preferred_element_type=jnp.float32)
        + jnp.dot(pos_ref[...], w2_ref[...], preferred_element_type=jnp.float32)
        + b_ref[...])

    lo = seg_lo_ref[i]
    hi = seg_hi_ref[i]
    base = i * BLKR
    iota_g = jax.lax.broadcasted_iota(jnp.int32, (G, 1), 0)
    neg_g = jnp.full((G, D), NEG, dtype=jnp.float32)

    def body(s, carry):
        rs = jnp.clip(starts_ref[s] - base, 0, BLKR)
        re = jnp.clip(starts_ref[s + 1] - base, 0, BLKR)

        gl = jnp.minimum(rs // G, NG - 1)
        gr = jnp.minimum(jnp.maximum(re - 1, 0) // G, NG - 1)
        il = iota_g + G * gl
        ir = iota_g + G * gr
        el = jnp.where((il >= rs) & (il < re), h_ref[pl.ds(G * gl, G), :],
                       neg_g)
        er = jnp.where((ir >= rs) & (ir < re), h_ref[pl.ds(G * gr, G), :],
                       neg_g)

        gi0 = (rs + G - 1) // G
        gi1 = re // G
        n2 = jnp.maximum(gi1 - gi0, 0) // 2

        def body2(k, accs):
            g = gi0 + 2 * k
            a0, a1 = accs
            return (jnp.maximum(a0, h_ref[pl.ds(G * g, G), :]),
                    jnp.maximum(a1, h_ref[pl.ds(G * (g + 1), G), :]))

        a0, a1 = jax.lax.fori_loop(0, n2, body2, (el, er))

        def body1(g, acc):
            return jnp.maximum(acc, h_ref[pl.ds(G * g, G), :])

        a0 = jax.lax.fori_loop(gi0 + 2 * n2, gi1, body1, a0)

        m = jnp.max(jnp.maximum(a0, a1), axis=0, keepdims=True)
        out_ref[pl.ds(s, 1), :] = jnp.maximum(out_ref[pl.ds(s, 1), :], m)
        return carry

    jax.lax.fori_loop(lo, hi + 1, body, 0)

    @pl.when(i == NSTEP - 1)
    def _final_relu():
        o = out_ref[...]
        out_ref[...] = jnp.where(o != NEG, jnp.maximum(o, 0.0), o)


def _segment_tables(ids32):
    q = jnp.arange(S + 1, dtype=jnp.int32)
    chunk_last = ids32[CH - 1::CH]
    coarse = jnp.sum(chunk_last[None, :] < q[:, None], axis=1,
                     dtype=jnp.int32)
    p = jnp.minimum(coarse, NCH - 1)
    rows = ids32.reshape(NCH, CH)[p]
    fine = jnp.sum(rows < q[:, None], axis=1, dtype=jnp.int32)
    starts = CH * p + fine

    bases = jnp.arange(NSTEP, dtype=jnp.int32) * BLKR
    seg_lo = jnp.sum(starts[None, :] <= bases[:, None], axis=1,
                     dtype=jnp.int32) - 1
    seg_hi = jnp.sum(starts[None, :] <= (bases + BLKR - 1)[:, None],
                     axis=1, dtype=jnp.int32) - 1
    return seg_lo, seg_hi, starts


def kernel(x, pos, batch, W, b):
    ids32 = batch.astype(jnp.int32)
    seg_lo, seg_hi, starts = _segment_tables(ids32)
    w1 = W[:D]
    w2 = W[D:]
    b2 = b.reshape(1, D)

    grid_spec = pltpu.PrefetchScalarGridSpec(
        num_scalar_prefetch=3,
        grid=(NSTEP,),
        in_specs=[
            pl.BlockSpec((BLKR, D), lambda i, *_: (i, 0)),
            pl.BlockSpec((BLKR, 3), lambda i, *_: (i, 0)),
            pl.BlockSpec((D, D), lambda i, *_: (0, 0)),
            pl.BlockSpec((3, D), lambda i, *_: (0, 0)),
            pl.BlockSpec((1, D), lambda i, *_: (0, 0)),
        ],
        out_specs=pl.BlockSpec((S, D), lambda i, *_: (0, 0)),
        scratch_shapes=[pltpu.VMEM((BLKR, D), jnp.float32)],
    )

    pooled = pl.pallas_call(
        _fused_kernel,
        grid_spec=grid_spec,
        out_shape=jax.ShapeDtypeStruct((S, D), jnp.float32),
    )(seg_lo, seg_hi, starts, x, pos, w1, w2, b2)

    pos_new = jnp.zeros((S, 3), dtype=x.dtype)
    batch_new = jnp.arange(S, dtype=batch.dtype)
    return (pooled, pos_new, batch_new)
